# Initial kernel scaffold; baseline (speedup 1.0000x reference)
import jax
import jax.numpy as jnp
from jax.experimental import pallas as pl


def kernel(x, edge_index, in_a, out_a, W_in, b_in, W_out, b_out):
    raise NotImplementedError("write your pallas kernel here")


# speedup vs baseline: 5.2970x; 5.2970x over previous
import functools

import jax
import jax.numpy as jnp
from jax import lax
from jax.experimental import pallas as pl
from jax.experimental.pallas import tpu as pltpu
from jax.experimental.pallas import tpu_sc as plsc

N = 10000
E = 320000
F = 128

NC = 2
NS = 16
NW = NC * NS
EP = E // NW
K = 80
CHUNKS = EP // K
N_PAD = 10240
ROWS_PER_TILE = N_PAD // NS


def _sc_segment_sum(x, src, dst, zeros):
    mesh = plsc.VectorSubcoreMesh(core_axis_name="c", subcore_axis_name="s",
                                  num_cores=NC, num_subcores=NS)

    @functools.partial(
        pl.kernel,
        out_type=jax.ShapeDtypeStruct((NC * N_PAD, F), jnp.float32),
        mesh=mesh,
        scratch_types=[
            pltpu.VMEM((K,), jnp.int32),
            pltpu.VMEM((K,), jnp.int32),
            pltpu.VMEM((K, F), jnp.float32),
            pltpu.VMEM_SHARED((N_PAD, F), jnp.float32),
            pltpu.SemaphoreType.DMA,
        ],
    )
    def seg_kernel(x_hbm, src_hbm, dst_hbm, zeros_hbm, out_hbm,
                   sidx, didx, rows, acc, sem):
        cid = lax.axis_index("c")
        sid = lax.axis_index("s")
        wid = cid * NS + sid

        pltpu.sync_copy(zeros_hbm, acc.at[pl.ds(sid * ROWS_PER_TILE,
                                                ROWS_PER_TILE)])
        plsc.subcore_barrier()

        def body(i, _):
            base = wid * EP + i * K
            pltpu.sync_copy(src_hbm.at[pl.ds(base, K)], sidx)
            pltpu.sync_copy(dst_hbm.at[pl.ds(base, K)], didx)
            pltpu.async_copy(x_hbm.at[sidx], rows, sem).wait()
            pltpu.sync_copy(rows, acc.at[didx], add=True)
            return 0

        lax.fori_loop(0, CHUNKS, body, 0)
        plsc.subcore_barrier()

        row0 = sid * ROWS_PER_TILE
        pltpu.sync_copy(acc.at[pl.ds(row0, ROWS_PER_TILE)],
                        out_hbm.at[pl.ds(cid * N_PAD + row0, ROWS_PER_TILE)])

    return seg_kernel(x, src, dst, zeros).reshape(NC, N_PAD, F)


def _tc_linear(x, parts, in_a, out_a, Wx, Ws, b):
    B = 1000
    grid = N // B

    def body(x_ref, p_ref, ina_ref, outa_ref, wx_ref, ws_ref, b_ref, o_ref):
        seg = p_ref[0] + p_ref[1]
        res = (jnp.dot(x_ref[...], wx_ref[...],
                       preferred_element_type=jnp.float32,
                       precision=lax.Precision.HIGHEST)
               + jnp.dot(seg, ws_ref[...],
                         preferred_element_type=jnp.float32,
                         precision=lax.Precision.HIGHEST)
               + b_ref[...])
        o_ref[...] = ina_ref[...] * res[:, :F] + outa_ref[...] * res[:, F:]

    return pl.pallas_call(
        body,
        grid=(grid,),
        in_specs=[
            pl.BlockSpec((B, F), lambda i: (i, 0)),
            pl.BlockSpec((NC, B, F), lambda i: (0, i, 0)),
            pl.BlockSpec((B, 1), lambda i: (i, 0)),
            pl.BlockSpec((B, 1), lambda i: (i, 0)),
            pl.BlockSpec((F, 2 * F), lambda i: (0, 0)),
            pl.BlockSpec((F, 2 * F), lambda i: (0, 0)),
            pl.BlockSpec((1, 2 * F), lambda i: (0, 0)),
        ],
        out_specs=pl.BlockSpec((B, F), lambda i: (i, 0)),
        out_shape=jax.ShapeDtypeStruct((N, F), jnp.float32),
    )(x, parts, in_a, out_a, Wx, Ws, b)


def kernel(x, edge_index, in_a, out_a, W_in, b_in, W_out, b_out):
    src = edge_index[0]
    dst = edge_index[1]
    zeros = jnp.zeros((ROWS_PER_TILE, F), jnp.float32)
    parts = _sc_segment_sum(x, src, dst, zeros)

    Wx = jnp.concatenate([W_in[:, :F].T, W_out[:, :F].T], axis=1)
    Ws = jnp.concatenate([W_in[:, F:].T, W_out[:, F:].T], axis=1)
    b = jnp.concatenate([b_in, b_out]).reshape(1, 2 * F)
    return _tc_linear(x, parts, in_a, out_a, Wx, Ws, b)


# speedup vs baseline: 7.5784x; 1.4307x over previous
import functools

import jax
import jax.numpy as jnp
from jax import lax
from jax.experimental import pallas as pl
from jax.experimental.pallas import tpu as pltpu
from jax.experimental.pallas import tpu_sc as plsc

N = 10000
E = 320000
F = 128
H = F // 2

NC = 2
NS = 16
EPT = E // NS
K = 80
CHUNKS = EPT // K
N_PAD = 10240
ROWS_PER_TILE = N_PAD // NS


def _sc_segment_sum(x2, src_pair, dst3, zeros):
    mesh = plsc.VectorSubcoreMesh(core_axis_name="c", subcore_axis_name="s",
                                  num_cores=NC, num_subcores=NS)

    @functools.partial(
        pl.kernel,
        out_type=jax.ShapeDtypeStruct((NC * N_PAD, H), jnp.float32),
        mesh=mesh,
        scratch_types=[
            pltpu.VMEM((CHUNKS, K), jnp.int32),
            pltpu.VMEM((CHUNKS, K), jnp.int32),
            pltpu.VMEM((K, H), jnp.float32),
            pltpu.VMEM((K, H), jnp.float32),
            pltpu.VMEM_SHARED((N_PAD, H), jnp.float32),
            pltpu.SemaphoreType.DMA,
            pltpu.SemaphoreType.DMA,
        ],
        compiler_params=pltpu.CompilerParams(use_tc_tiling_on_sc=False),
    )
    def seg_kernel(x_hbm, src_hbm, dst_hbm, zeros_hbm, out_hbm,
                   sidx, didx, rows0, rows1, acc, sem0, sem1):
        cid = lax.axis_index("c")
        sid = lax.axis_index("s")

        pltpu.sync_copy(src_hbm.at[cid, sid], sidx)
        pltpu.sync_copy(dst_hbm.at[sid], didx)
        pltpu.sync_copy(zeros_hbm, acc.at[pl.ds(sid * ROWS_PER_TILE,
                                                ROWS_PER_TILE)])
        plsc.subcore_barrier()

        pltpu.async_copy(x_hbm.at[sidx.at[0]], rows0, sem0)
        pltpu.async_copy(x_hbm.at[sidx.at[1]], rows1, sem1)

        def step(c, buf, sem):
            pltpu.make_async_copy(x_hbm.at[pl.ds(0, K)], buf, sem).wait()
            pltpu.sync_copy(buf, acc.at[didx.at[c]], add=True)
            nxt = jnp.minimum(c + 2, CHUNKS - 1)
            pltpu.async_copy(x_hbm.at[sidx.at[nxt]], buf, sem)

        def body(j, _):
            step(2 * j, rows0, sem0)
            step(2 * j + 1, rows1, sem1)
            return 0

        lax.fori_loop(0, CHUNKS // 2, body, 0)
        pltpu.make_async_copy(x_hbm.at[pl.ds(0, K)], rows0, sem0).wait()
        pltpu.make_async_copy(x_hbm.at[pl.ds(0, K)], rows1, sem1).wait()
        plsc.subcore_barrier()

        row0 = sid * ROWS_PER_TILE
        pltpu.sync_copy(acc.at[pl.ds(row0, ROWS_PER_TILE)],
                        out_hbm.at[pl.ds(cid * N_PAD + row0, ROWS_PER_TILE)])

    return seg_kernel(x2, src_pair, dst3, zeros)


def _tc_linear(x, parts, in_a, out_a, Wx, Ws0, Ws1, b):
    B = 1000
    grid = N // B

    def body(x_ref, p_ref, ina_ref, outa_ref, wx_ref, ws0_ref, ws1_ref,
             b_ref, o_ref):
        mm = functools.partial(jnp.dot, preferred_element_type=jnp.float32,
                               precision=lax.Precision.HIGHEST)
        res = (mm(x_ref[...], wx_ref[...])
               + mm(p_ref[0], ws0_ref[...])
               + mm(p_ref[1], ws1_ref[...])
               + b_ref[...])
        o_ref[...] = ina_ref[...] * res[:, :F] + outa_ref[...] * res[:, F:]

    return pl.pallas_call(
        body,
        grid=(grid,),
        in_specs=[
            pl.BlockSpec((B, F), lambda i: (i, 0)),
            pl.BlockSpec((NC, B, H), lambda i: (0, i, 0)),
            pl.BlockSpec((B, 1), lambda i: (i, 0)),
            pl.BlockSpec((B, 1), lambda i: (i, 0)),
            pl.BlockSpec((F, 2 * F), lambda i: (0, 0)),
            pl.BlockSpec((H, 2 * F), lambda i: (0, 0)),
            pl.BlockSpec((H, 2 * F), lambda i: (0, 0)),
            pl.BlockSpec((1, 2 * F), lambda i: (0, 0)),
        ],
        out_specs=pl.BlockSpec((B, F), lambda i: (i, 0)),
        out_shape=jax.ShapeDtypeStruct((N, F), jnp.float32),
    )(x, parts, in_a, out_a, Wx, Ws0, Ws1, b)


def kernel(x, edge_index, in_a, out_a, W_in, b_in, W_out, b_out):
    src = edge_index[0]
    dst = edge_index[1]
    x2 = x.reshape(N, NC, H).swapaxes(0, 1).reshape(NC * N, H)
    src_pair = jnp.stack([src, src + N]).reshape(NC, NS, CHUNKS, K)
    dst3 = dst.reshape(NS, CHUNKS, K)
    zeros = jnp.zeros((ROWS_PER_TILE, H), jnp.float32)
    parts = _sc_segment_sum(x2, src_pair, dst3, zeros).reshape(NC, N_PAD, H)

    Wx = jnp.concatenate([W_in[:, :F].T, W_out[:, :F].T], axis=1)
    Ws = jnp.concatenate([W_in[:, F:].T, W_out[:, F:].T], axis=1)
    b = jnp.concatenate([b_in, b_out]).reshape(1, 2 * F)
    return _tc_linear(x, parts, in_a, out_a, Wx, Ws[:H], Ws[H:], b)
